# Initial kernel scaffold; baseline (speedup 1.0000x reference)
import jax
import jax.numpy as jnp
from jax.experimental import pallas as pl


def kernel(x_u, x_v, support_vals, weights_u, weights_v, support_rows, support_cols):
    raise NotImplementedError("write your pallas kernel here")


# speedup vs baseline: 3.4148x; 3.4148x over previous
import functools

import jax
import jax.numpy as jnp
from jax import lax
from jax.experimental import pallas as pl
from jax.experimental.pallas import tpu as pltpu
from jax.experimental.pallas import tpu_sc as plsc

L = 16
CH = 128
NT = 16


def _mm_body(ns, xu_ref, xv_ref, wu_ref, wv_ref, ou_ref, ov_ref):
    i = pl.program_id(0)

    def cum(w_ref):
        acc = jnp.zeros(w_ref.shape[1:], jnp.float32)
        for j in range(ns):
            acc = acc + jnp.where(j <= i, w_ref[j], 0.0)
        return acc

    ou_ref[0] = jnp.dot(xu_ref[...], cum(wu_ref), preferred_element_type=jnp.float32)
    ov_ref[0] = jnp.dot(xv_ref[...], cum(wv_ref), preferred_element_type=jnp.float32)


def _dense_tmp(x_u, x_v, weights_u, weights_v):
    ns, di, do = weights_u.shape
    n_u = x_u.shape[0]
    n_v = x_v.shape[0]
    b = 1000
    grid = (ns, n_u // b)
    return pl.pallas_call(
        functools.partial(_mm_body, ns),
        grid=grid,
        in_specs=[
            pl.BlockSpec((b, di), lambda i, k: (k, 0)),
            pl.BlockSpec((b, di), lambda i, k: (k, 0)),
            pl.BlockSpec((ns, di, do), lambda i, k: (0, 0, 0)),
            pl.BlockSpec((ns, di, do), lambda i, k: (0, 0, 0)),
        ],
        out_specs=[
            pl.BlockSpec((1, b, do), lambda i, k: (i, k, 0)),
            pl.BlockSpec((1, b, do), lambda i, k: (i, k, 0)),
        ],
        out_shape=[
            jax.ShapeDtypeStruct((ns, n_u, do), jnp.float32),
            jax.ShapeDtypeStruct((ns, n_v, do), jnp.float32),
        ],
    )(x_u, x_v, weights_u, weights_v)


def _prep_edges(gidx, sidx, vals, table_rows):
    ns, nnz = gidx.shape
    g = (gidx + (jnp.arange(ns, dtype=jnp.int32) * table_rows)[:, None]).reshape(-1)
    s = sidx.reshape(-1)
    v = vals.reshape(-1)
    e = ns * nnz
    per_tile = -(-e // (NT * CH)) * CH
    pad = NT * per_tile - e
    g = jnp.pad(g, (0, pad))
    s = jnp.pad(s, (0, pad))
    v = jnp.pad(v, (0, pad))
    return g, s, v, per_tile // CH


def kernel(x_u, x_v, support_vals, weights_u, weights_v, support_rows, support_cols):
    ns, di, do = weights_u.shape
    n_u = x_u.shape[0]
    n_v = x_v.shape[0]
    nvec = do // L

    tmp_u, tmp_v = _dense_tmp(x_u, x_v, weights_u, weights_v)
    tab_u = tmp_u.reshape(ns * n_u, do)
    tab_v = tmp_v.reshape(ns * n_v, do)

    rows = support_rows.astype(jnp.int32)
    cols = support_cols.astype(jnp.int32)
    g_u, s_u, v_u, nc = _prep_edges(cols, rows, support_vals, n_v)
    g_v, s_v, v_v, _ = _prep_edges(rows, cols, support_vals, n_u)

    wchunk = 16
    chunks_per_tile = (n_u // wchunk) // NT
    extra = (n_u // wchunk) - chunks_per_tile * NT

    mesh = plsc.VectorSubcoreMesh(core_axis_name="c", subcore_axis_name="s")

    @functools.partial(
        pl.kernel,
        mesh=mesh,
        out_type=[
            jax.ShapeDtypeStruct((n_u, do), jnp.float32),
            jax.ShapeDtypeStruct((n_v, do), jnp.float32),
        ],
        scratch_types=[
            pltpu.VMEM_SHARED((n_u, do), jnp.float32),
            pltpu.VMEM((CH,), jnp.int32),
            pltpu.VMEM((CH,), jnp.int32),
            pltpu.VMEM((CH,), jnp.float32),
            pltpu.VMEM((CH, do), jnp.float32),
            pltpu.SemaphoreType.DMA,
        ],
    )
    def sc_fn(tabv_h, tabu_h, gu_h, su_h, vu_h, gv_h, sv_h, vv_h, zu_h, zv_h,
              acc, gbuf, sbuf, vbuf, rowbuf, sem):
        c = lax.axis_index("c")
        t = lax.axis_index("s")
        my_chunks = chunks_per_tile + jnp.where(t == NT - 1, extra, 0)
        cbase = t * chunks_per_tile

        def zrow(e, _):
            for j in range(nvec):
                rowbuf[e, pl.ds(j * L, L)] = jnp.zeros((L,), jnp.float32)
            return 0

        lax.fori_loop(0, wchunk, zrow, 0)

        def zchunk(q, _):
            pltpu.sync_copy(rowbuf.at[pl.ds(0, wchunk)],
                            acc.at[pl.ds((cbase + q) * wchunk, wchunk)])
            return 0

        lax.fori_loop(0, my_chunks, zchunk, 0)
        plsc.subcore_barrier()

        def side(tab_h, g_h, s_h, v_h):
            def chunk(k, _):
                ebase = (t * nc + k) * CH
                pltpu.sync_copy(g_h.at[pl.ds(ebase, CH)], gbuf)
                pltpu.sync_copy(s_h.at[pl.ds(ebase, CH)], sbuf)
                pltpu.sync_copy(v_h.at[pl.ds(ebase, CH)], vbuf)
                pltpu.async_copy(tab_h.at[gbuf], rowbuf, sem).wait()

                def scale(g, _):
                    vals16 = vbuf[pl.ds(g * L, L)]
                    for lane in range(L):
                        val = vals16[lane]
                        e = g * L + lane
                        for j in range(nvec):
                            sl = pl.ds(j * L, L)
                            rowbuf[e, sl] = rowbuf[e, sl] * val
                    return 0

                lax.fori_loop(0, CH // L, scale, 0)
                pltpu.sync_copy(rowbuf, acc.at[sbuf], add=True)
                return 0

            lax.fori_loop(0, nc, chunk, 0)

        pl.when(c == 0)(lambda: side(tabv_h, gu_h, su_h, vu_h))
        pl.when(c == 1)(lambda: side(tabu_h, gv_h, sv_h, vv_h))
        plsc.subcore_barrier()

        def writeout(out_h):
            def wout(q, _):
                off = (cbase + q) * wchunk
                pltpu.sync_copy(acc.at[pl.ds(off, wchunk)],
                                rowbuf.at[pl.ds(0, wchunk)])

                def relu(e, _):
                    for j in range(nvec):
                        sl = pl.ds(j * L, L)
                        rowbuf[e, sl] = jnp.maximum(rowbuf[e, sl], 0.0)
                    return 0

                lax.fori_loop(0, wchunk, relu, 0)
                pltpu.sync_copy(rowbuf.at[pl.ds(0, wchunk)],
                                out_h.at[pl.ds(off, wchunk)])
                return 0

            lax.fori_loop(0, my_chunks, wout, 0)

        pl.when(c == 0)(lambda: writeout(zu_h))
        pl.when(c == 1)(lambda: writeout(zv_h))

    z_u, z_v = sc_fn(tab_v, tab_u, g_u, s_u, v_u, g_v, s_v, v_v)
    return z_u, z_v
